# baseline (speedup 1.0000x reference)
import functools

import jax
import jax.numpy as jnp
from jax import lax
from jax.experimental import pallas as pl
from jax.experimental.pallas import tpu as pltpu
from jax.experimental.pallas import tpu_sc as plsc

B = 16384
L = 50
VOCAB = 1000000
D = 64

NC = 2
NS = 16
NW = NC * NS
LANES = 16
ND = D // LANES
WROW = 2 * D

BAGS_PER_W = B // NW
G = 8
NCHUNK = BAGS_PER_W // G
NBUF = 2
NT = (G * L + LANES - 1) // LANES

VB = 8192
GRID = 62
NP = GRID * VB
IN_BLOCKS = -(-VOCAB // VB) - 1


def _pack_kernel(x1_ref, x2_ref, o_ref):
    o_ref[:, 0:D] = x1_ref[...].T
    o_ref[:, D:WROW] = x2_ref[...].T


def _bag_kernel(fh_hbm, fw_hbm, tab_hbm, out_hbm,
                idx_v, idx2_v, rows_v, fw_v, out_v, sems):
    wid = lax.axis_index("s") * NC + lax.axis_index("c")
    bag0 = wid * BAGS_PER_W
    lane = lax.iota(jnp.int32, LANES)

    def issue(buf, c):
        base = bag0 + c * G
        pltpu.sync_copy(fh_hbm.at[pl.ds(base, G), :], idx_v.at[buf])
        pltpu.sync_copy(fw_hbm.at[pl.ds(base, G), :], fw_v.at[buf])
        for m in range(NT):
            j = lane + m * LANES
            r = j // L
            col = j - r * L
            r = jnp.minimum(r, G - 1)
            v = plsc.load_gather(idx_v.at[buf], [r, col])
            v2 = jnp.where(v >= NP, 2 * (v - NP) + 1, 2 * v)
            r2 = j // (2 * L)
            c2 = j - r2 * (2 * L)
            r2 = jnp.minimum(r2, G // 2 - 1)
            plsc.store_scatter(idx2_v.at[buf], [r2, c2], v2)
        for q in range(G // 2):
            pltpu.async_copy(tab_hbm.at[idx2_v.at[buf, q]], rows_v.at[buf, q],
                             sems.at[buf])

    def drain(buf):
        for q in range(G // 2):
            pltpu.make_async_copy(tab_hbm.at[idx2_v.at[buf, q]],
                                  rows_v.at[buf, q], sems.at[buf]).wait()

    def compute(buf, c):
        for b in range(G):
            bag_rows = rows_v.at[buf, b // 2]
            bag_fw = fw_v.at[buf, b]
            off = (b % 2) * L

            accs = (jnp.zeros((LANES,), jnp.float32),) * ND
            for m in range(4):
                w16 = plsc.load_gather(
                    bag_fw, [jnp.minimum(lane + m * LANES, L - 1)])
                cnt = min(LANES, L - m * LANES)

                def body(l2, accs, m=m, w16=w16):
                    lsp = jnp.full((LANES,), off + m * LANES + l2, jnp.int32)
                    w = jnp.take(w16, jnp.full((LANES,), l2, jnp.int32))
                    return tuple(
                        accs[k] + plsc.load_gather(
                            bag_rows, [lsp, lane + k * LANES]) * w
                        for k in range(ND))

                accs = lax.fori_loop(0, cnt, body, accs, unroll=4)

            ss = accs[0] * accs[0]
            for k in range(1, ND):
                ss = ss + accs[k] * accs[k]
            sv = ss
            for shift in (8, 4, 2, 1):
                sv = sv + jnp.take(sv, lane ^ shift)
            i = plsc.bitcast(sv, jnp.int32)
            y = plsc.bitcast(jnp.int32(0x5F3759DF) - (i >> 1), jnp.float32)
            for _ in range(3):
                y = y * (1.5 - 0.5 * sv * y * y)
            y = jnp.minimum(y, 1e12)
            for k in range(ND):
                out_v[b, pl.ds(k * LANES, LANES)] = accs[k] * y
        pltpu.sync_copy(out_v, out_hbm.at[pl.ds(bag0 + c * G, G), :])

    issue(0, 0)

    @pl.loop(0, NCHUNK, step=NBUF)
    def _(c0):
        for p in range(NBUF):
            c = c0 + p

            @pl.when(c + 1 < NCHUNK)
            def _():
                issue((p + 1) % NBUF, c + 1)

            drain(p)
            compute(p, c)


@jax.jit
def kernel(feature_hashes, feature_weights, weight):
    mesh = plsc.VectorSubcoreMesh(core_axis_name="c", subcore_axis_name="s")
    f = pl.kernel(
        _bag_kernel,
        out_type=jax.ShapeDtypeStruct((B, D), jnp.float32),
        mesh=mesh,
        compiler_params=pltpu.CompilerParams(
            needs_layout_passes=False, use_tc_tiling_on_sc=False),
        scratch_types=[
            pltpu.VMEM((NBUF, G, L), jnp.int32),
            pltpu.VMEM((NBUF, G // 2, 2 * L), jnp.int32),
            pltpu.VMEM((NBUF, G // 2, 2 * L, D), jnp.float32),
            pltpu.VMEM((NBUF, G, L), jnp.float32),
            pltpu.VMEM((G, D), jnp.float32),
            pltpu.SemaphoreType.DMA((NBUF,)),
        ],
    )
    packed = pl.pallas_call(
        _pack_kernel,
        grid=(GRID,),
        in_specs=[
            pl.BlockSpec((D, VB), lambda j: (0, j)),
            pl.BlockSpec(
                (D, VB),
                lambda j: (0, jnp.minimum(j + GRID, IN_BLOCKS))),
        ],
        out_specs=pl.BlockSpec((VB, WROW), lambda j: (j, 0)),
        out_shape=jax.ShapeDtypeStruct((NP, WROW), jnp.float32),
    )(weight.T, weight.T)
    flat = packed.reshape(2 * NP, D)
    return f(feature_hashes.astype(jnp.int32), feature_weights, flat)


# speedup vs baseline: 1.0705x; 1.0705x over previous
import functools

import jax
import jax.numpy as jnp
from jax import lax
from jax.experimental import pallas as pl
from jax.experimental.pallas import tpu as pltpu
from jax.experimental.pallas import tpu_sc as plsc

B = 16384
L = 50
VOCAB = 1000000
D = 64

NC = 2
NS = 16
NW = NC * NS
LANES = 16
ND = D // LANES
WROW = 2 * D

BAGS_PER_W = B // NW
G = 8
NCHUNK = BAGS_PER_W // G
NBUF = 2
NT = (G * L + LANES - 1) // LANES

VB = 8192
GRID = 62
NP = GRID * VB
IN_BLOCKS = -(-VOCAB // VB) - 1


def _pack_kernel(x1_ref, x2_ref, o_ref):
    o_ref[:, 0:D] = x1_ref[...].T
    o_ref[:, D:WROW] = x2_ref[...].T


def _bag_kernel(fh_hbm, fw_hbm, tab_hbm, out_hbm,
                idx_v, idx2_v, rows_v, fw_v, out_v, sems):
    wid = lax.axis_index("s") * NC + lax.axis_index("c")
    bag0 = wid * BAGS_PER_W
    lane = lax.iota(jnp.int32, LANES)

    def issue(buf, c):
        base = bag0 + c * G
        pltpu.sync_copy(fh_hbm.at[pl.ds(base, G), :], idx_v.at[buf])
        pltpu.sync_copy(fw_hbm.at[pl.ds(base, G), :], fw_v.at[buf])
        for m in range(NT):
            j = lane + m * LANES
            r = j // L
            col = j - r * L
            r = jnp.minimum(r, G - 1)
            v = plsc.load_gather(idx_v.at[buf], [r, col])
            v2 = jnp.where(v >= NP, 2 * (v - NP) + 1, 2 * v)
            r2 = j // (2 * L)
            c2 = j - r2 * (2 * L)
            r2 = jnp.minimum(r2, G // 2 - 1)
            plsc.store_scatter(idx2_v.at[buf], [r2, c2], v2)
        for q in range(G // 2):
            pltpu.async_copy(tab_hbm.at[idx2_v.at[buf, q]], rows_v.at[buf, q],
                             sems.at[buf])

    def drain(buf):
        for q in range(G // 2):
            pltpu.make_async_copy(tab_hbm.at[idx2_v.at[buf, q]],
                                  rows_v.at[buf, q], sems.at[buf]).wait()

    def compute(buf, c):
        for b in range(G):
            bag_rows = rows_v.at[buf, b // 2]
            bag_fw = fw_v.at[buf, b]
            off = (b % 2) * L

            accs = (jnp.zeros((LANES,), jnp.float32),) * ND
            for m in range(4):
                w16 = plsc.load_gather(
                    bag_fw, [jnp.minimum(lane + m * LANES, L - 1)])
                cnt = min(LANES, L - m * LANES)

                def body(l2, accs, m=m, w16=w16):
                    lsp = jnp.full((LANES,), off + m * LANES + l2, jnp.int32)
                    w = jnp.take(w16, jnp.full((LANES,), l2, jnp.int32))
                    return tuple(
                        accs[k] + plsc.load_gather(
                            bag_rows, [lsp, lane + k * LANES]) * w
                        for k in range(ND))

                accs = lax.fori_loop(0, cnt, body, accs, unroll=2)

            ss = accs[0] * accs[0]
            for k in range(1, ND):
                ss = ss + accs[k] * accs[k]
            sv = ss
            for shift in (8, 4, 2, 1):
                sv = sv + jnp.take(sv, lane ^ shift)
            i = plsc.bitcast(sv, jnp.int32)
            y = plsc.bitcast(jnp.int32(0x5F3759DF) - (i >> 1), jnp.float32)
            for _ in range(3):
                y = y * (1.5 - 0.5 * sv * y * y)
            y = jnp.minimum(y, 1e12)
            for k in range(ND):
                out_v[b, pl.ds(k * LANES, LANES)] = accs[k] * y
        pltpu.sync_copy(out_v, out_hbm.at[pl.ds(bag0 + c * G, G), :])

    issue(0, 0)

    @pl.loop(0, NCHUNK, step=NBUF)
    def _(c0):
        for p in range(NBUF):
            c = c0 + p

            @pl.when(c + 1 < NCHUNK)
            def _():
                issue((p + 1) % NBUF, c + 1)

            drain(p)
            compute(p, c)


@jax.jit
def kernel(feature_hashes, feature_weights, weight):
    mesh = plsc.VectorSubcoreMesh(core_axis_name="c", subcore_axis_name="s")
    f = pl.kernel(
        _bag_kernel,
        out_type=jax.ShapeDtypeStruct((B, D), jnp.float32),
        mesh=mesh,
        compiler_params=pltpu.CompilerParams(
            needs_layout_passes=False, use_tc_tiling_on_sc=False),
        scratch_types=[
            pltpu.VMEM((NBUF, G, L), jnp.int32),
            pltpu.VMEM((NBUF, G // 2, 2 * L), jnp.int32),
            pltpu.VMEM((NBUF, G // 2, 2 * L, D), jnp.float32),
            pltpu.VMEM((NBUF, G, L), jnp.float32),
            pltpu.VMEM((G, D), jnp.float32),
            pltpu.SemaphoreType.DMA((NBUF,)),
        ],
    )
    packed = pl.pallas_call(
        _pack_kernel,
        grid=(GRID,),
        in_specs=[
            pl.BlockSpec((D, VB), lambda j: (0, j)),
            pl.BlockSpec(
                (D, VB),
                lambda j: (0, jnp.minimum(j + GRID, IN_BLOCKS))),
        ],
        out_specs=pl.BlockSpec((VB, WROW), lambda j: (j, 0)),
        out_shape=jax.ShapeDtypeStruct((NP, WROW), jnp.float32),
    )(weight.T, weight.T)
    flat = packed.reshape(2 * NP, D)
    return f(feature_hashes.astype(jnp.int32), feature_weights, flat)


# speedup vs baseline: 1.1020x; 1.0294x over previous
import functools

import jax
import jax.numpy as jnp
from jax import lax
from jax.experimental import pallas as pl
from jax.experimental.pallas import tpu as pltpu
from jax.experimental.pallas import tpu_sc as plsc

B = 16384
L = 50
VOCAB = 1000000
D = 64

NC = 2
NS = 16
NW = NC * NS
LANES = 16
ND = D // LANES
WROW = 2 * D

BAGS_PER_W = B // NW
G = 8
NCHUNK = BAGS_PER_W // G
NBUF = 2
NT = (G * L + LANES - 1) // LANES

VB = 8192
GRID = 62
NP = GRID * VB
IN_BLOCKS = -(-VOCAB // VB) - 1


def _pack_kernel(x1_ref, x2_ref, o_ref):
    o_ref[:, 0:D] = x1_ref[...].T
    o_ref[:, D:WROW] = x2_ref[...].T


def _bag_kernel(fh_hbm, fw_hbm, tab_hbm, out_hbm,
                idx_v, idx2_v, rows_v, fw_v, out_v, sems):
    wid = lax.axis_index("s") * NC + lax.axis_index("c")
    bag0 = wid * BAGS_PER_W
    lane = lax.iota(jnp.int32, LANES)

    def issue(buf, c):
        base = bag0 + c * G
        pltpu.sync_copy(fh_hbm.at[pl.ds(base, G), :], idx_v.at[buf])
        pltpu.sync_copy(fw_hbm.at[pl.ds(base, G), :], fw_v.at[buf])
        for m in range(NT):
            j = lane + m * LANES
            r = j // L
            col = j - r * L
            r = jnp.minimum(r, G - 1)
            v = plsc.load_gather(idx_v.at[buf], [r, col])
            v2 = jnp.where(v >= NP, 2 * (v - NP) + 1, 2 * v)
            r2 = j // (2 * L)
            c2 = j - r2 * (2 * L)
            r2 = jnp.minimum(r2, G // 2 - 1)
            plsc.store_scatter(idx2_v.at[buf], [r2, c2], v2)
        for q in range(G // 2):
            pltpu.async_copy(tab_hbm.at[idx2_v.at[buf, q]], rows_v.at[buf, q],
                             sems.at[buf])

    def drain(buf):
        for q in range(G // 2):
            pltpu.make_async_copy(tab_hbm.at[idx2_v.at[buf, q]],
                                  rows_v.at[buf, q], sems.at[buf]).wait()

    def compute(buf, c):
        for b in range(G):
            bag_rows = rows_v.at[buf, b // 2]
            bag_fw = fw_v.at[buf, b]
            off = (b % 2) * L

            accs = (jnp.zeros((LANES,), jnp.float32),) * ND
            for m in range(4):
                w16 = plsc.load_gather(
                    bag_fw, [jnp.minimum(lane + m * LANES, L - 1)])
                cnt = min(LANES, L - m * LANES)

                def body(l2, accs, m=m, w16=w16):
                    lsp = jnp.full((LANES,), off + m * LANES + l2, jnp.int32)
                    w = jnp.take(w16, jnp.full((LANES,), l2, jnp.int32))
                    return tuple(
                        accs[k] + plsc.load_gather(
                            bag_rows, [lsp, lane + k * LANES]) * w
                        for k in range(ND))

                accs = lax.fori_loop(0, cnt, body, accs)

            ss = accs[0] * accs[0]
            for k in range(1, ND):
                ss = ss + accs[k] * accs[k]
            sv = ss
            for shift in (8, 4, 2, 1):
                sv = sv + jnp.take(sv, lane ^ shift)
            i = plsc.bitcast(sv, jnp.int32)
            y = plsc.bitcast(jnp.int32(0x5F3759DF) - (i >> 1), jnp.float32)
            for _ in range(3):
                y = y * (1.5 - 0.5 * sv * y * y)
            y = jnp.minimum(y, 1e12)
            for k in range(ND):
                out_v[b, pl.ds(k * LANES, LANES)] = accs[k] * y
        pltpu.sync_copy(out_v, out_hbm.at[pl.ds(bag0 + c * G, G), :])

    issue(0, 0)

    @pl.loop(0, NCHUNK, step=NBUF)
    def _(c0):
        for p in range(NBUF):
            c = c0 + p

            @pl.when(c + 1 < NCHUNK)
            def _():
                issue((p + 1) % NBUF, c + 1)

            drain(p)
            compute(p, c)


@jax.jit
def kernel(feature_hashes, feature_weights, weight):
    mesh = plsc.VectorSubcoreMesh(core_axis_name="c", subcore_axis_name="s")
    f = pl.kernel(
        _bag_kernel,
        out_type=jax.ShapeDtypeStruct((B, D), jnp.float32),
        mesh=mesh,
        compiler_params=pltpu.CompilerParams(
            needs_layout_passes=False, use_tc_tiling_on_sc=False),
        scratch_types=[
            pltpu.VMEM((NBUF, G, L), jnp.int32),
            pltpu.VMEM((NBUF, G // 2, 2 * L), jnp.int32),
            pltpu.VMEM((NBUF, G // 2, 2 * L, D), jnp.float32),
            pltpu.VMEM((NBUF, G, L), jnp.float32),
            pltpu.VMEM((G, D), jnp.float32),
            pltpu.SemaphoreType.DMA((NBUF,)),
        ],
    )
    packed = pl.pallas_call(
        _pack_kernel,
        grid=(GRID,),
        in_specs=[
            pl.BlockSpec((D, VB), lambda j: (0, j)),
            pl.BlockSpec(
                (D, VB),
                lambda j: (0, jnp.minimum(j + GRID, IN_BLOCKS))),
        ],
        out_specs=pl.BlockSpec((VB, WROW), lambda j: (j, 0)),
        out_shape=jax.ShapeDtypeStruct((NP, WROW), jnp.float32),
    )(weight.T, weight.T)
    flat = packed.reshape(2 * NP, D)
    return f(feature_hashes.astype(jnp.int32), feature_weights, flat)


# speedup vs baseline: 1.1343x; 1.0293x over previous
import functools

import jax
import jax.numpy as jnp
from jax import lax
from jax.experimental import pallas as pl
from jax.experimental.pallas import tpu as pltpu
from jax.experimental.pallas import tpu_sc as plsc

B = 16384
L = 50
VOCAB = 1000000
D = 64

NC = 2
NS = 16
NW = NC * NS
LANES = 16
ND = D // LANES
WROW = 2 * D

BAGS_PER_W = B // NW
G = 16
NCHUNK = BAGS_PER_W // G
NBUF = 2
NT = (G * L + LANES - 1) // LANES

VB = 8192
GRID = 62
NP = GRID * VB
IN_BLOCKS = -(-VOCAB // VB) - 1


def _pack_kernel(x1_ref, x2_ref, o_ref):
    o_ref[:, 0:D] = x1_ref[...].T
    o_ref[:, D:WROW] = x2_ref[...].T


def _bag_kernel(fh_hbm, fw_hbm, tab_hbm, out_hbm,
                idx_v, idx2_v, rows_v, fw_v, out_v, sems):
    wid = lax.axis_index("s") * NC + lax.axis_index("c")
    bag0 = wid * BAGS_PER_W
    lane = lax.iota(jnp.int32, LANES)

    def issue(buf, c):
        base = bag0 + c * G
        pltpu.sync_copy(fh_hbm.at[pl.ds(base, G), :], idx_v.at[buf])
        pltpu.sync_copy(fw_hbm.at[pl.ds(base, G), :], fw_v.at[buf])
        for m in range(NT):
            j = lane + m * LANES
            r = j // L
            col = j - r * L
            r = jnp.minimum(r, G - 1)
            v = plsc.load_gather(idx_v.at[buf], [r, col])
            v2 = jnp.where(v >= NP, 2 * (v - NP) + 1, 2 * v)
            r2 = j // (2 * L)
            c2 = j - r2 * (2 * L)
            r2 = jnp.minimum(r2, G // 2 - 1)
            plsc.store_scatter(idx2_v.at[buf], [r2, c2], v2)
        for q in range(G // 2):
            pltpu.async_copy(tab_hbm.at[idx2_v.at[buf, q]], rows_v.at[buf, q],
                             sems.at[buf])

    def drain(buf):
        for q in range(G // 2):
            pltpu.make_async_copy(tab_hbm.at[idx2_v.at[buf, q]],
                                  rows_v.at[buf, q], sems.at[buf]).wait()

    def compute(buf, c):
        for b in range(G):
            bag_rows = rows_v.at[buf, b // 2]
            bag_fw = fw_v.at[buf, b]
            off = (b % 2) * L

            accs = (jnp.zeros((LANES,), jnp.float32),) * ND
            for m in range(4):
                w16 = plsc.load_gather(
                    bag_fw, [jnp.minimum(lane + m * LANES, L - 1)])
                cnt = min(LANES, L - m * LANES)

                def body(l2, accs, m=m, w16=w16):
                    lsp = jnp.full((LANES,), off + m * LANES + l2, jnp.int32)
                    w = jnp.take(w16, jnp.full((LANES,), l2, jnp.int32))
                    return tuple(
                        accs[k] + plsc.load_gather(
                            bag_rows, [lsp, lane + k * LANES]) * w
                        for k in range(ND))

                accs = lax.fori_loop(0, cnt, body, accs)

            ss = accs[0] * accs[0]
            for k in range(1, ND):
                ss = ss + accs[k] * accs[k]
            sv = ss
            for shift in (8, 4, 2, 1):
                sv = sv + jnp.take(sv, lane ^ shift)
            i = plsc.bitcast(sv, jnp.int32)
            y = plsc.bitcast(jnp.int32(0x5F3759DF) - (i >> 1), jnp.float32)
            for _ in range(3):
                y = y * (1.5 - 0.5 * sv * y * y)
            y = jnp.minimum(y, 1e12)
            for k in range(ND):
                out_v[b, pl.ds(k * LANES, LANES)] = accs[k] * y
        pltpu.sync_copy(out_v, out_hbm.at[pl.ds(bag0 + c * G, G), :])

    issue(0, 0)

    @pl.loop(0, NCHUNK, step=NBUF)
    def _(c0):
        for p in range(NBUF):
            c = c0 + p

            @pl.when(c + 1 < NCHUNK)
            def _():
                issue((p + 1) % NBUF, c + 1)

            drain(p)
            compute(p, c)


@jax.jit
def kernel(feature_hashes, feature_weights, weight):
    mesh = plsc.VectorSubcoreMesh(core_axis_name="c", subcore_axis_name="s")
    f = pl.kernel(
        _bag_kernel,
        out_type=jax.ShapeDtypeStruct((B, D), jnp.float32),
        mesh=mesh,
        compiler_params=pltpu.CompilerParams(
            needs_layout_passes=False, use_tc_tiling_on_sc=False),
        scratch_types=[
            pltpu.VMEM((NBUF, G, L), jnp.int32),
            pltpu.VMEM((NBUF, G // 2, 2 * L), jnp.int32),
            pltpu.VMEM((NBUF, G // 2, 2 * L, D), jnp.float32),
            pltpu.VMEM((NBUF, G, L), jnp.float32),
            pltpu.VMEM((G, D), jnp.float32),
            pltpu.SemaphoreType.DMA((NBUF,)),
        ],
    )
    packed = pl.pallas_call(
        _pack_kernel,
        grid=(GRID,),
        in_specs=[
            pl.BlockSpec((D, VB), lambda j: (0, j)),
            pl.BlockSpec(
                (D, VB),
                lambda j: (0, jnp.minimum(j + GRID, IN_BLOCKS))),
        ],
        out_specs=pl.BlockSpec((VB, WROW), lambda j: (j, 0)),
        out_shape=jax.ShapeDtypeStruct((NP, WROW), jnp.float32),
    )(weight.T, weight.T)
    flat = packed.reshape(2 * NP, D)
    return f(feature_hashes.astype(jnp.int32), feature_weights, flat)
